# baseline (speedup 1.0000x reference)
import numpy as np
import jax
import jax.numpy as jnp
from jax.experimental import pallas as pl

_RATIOS = np.array([0.5, 1.0, 2.0])
_SCALES = np.array([2 ** 0, 2 ** (1.0 / 3.0), 2 ** (2.0 / 3.0)])

_LANES = 1152


def _anchor_sizes(box_size):
    anchors = box_size * np.tile(_SCALES, (2, len(_RATIOS))).T
    areas = anchors[:, 0] * anchors[:, 1]
    anchors[:, 0] = np.sqrt(areas * np.repeat(_RATIOS, len(_SCALES)))
    anchors[:, 1] = anchors[:, 0] / np.repeat(_RATIOS, len(_SCALES))
    return anchors.astype(np.float32)


def _build_tables():
    t = np.arange(_LANES)
    q = (t // 36).astype(np.int32)
    m36 = t % 36
    a = m36 // 4
    j = m36 % 4
    m0 = (j == 0).astype(np.float32)
    m1 = (j == 1).astype(np.float32)
    mx = (j % 2 == 0).astype(np.float32)
    my = (j % 2 == 1).astype(np.float32)

    levels = []
    for w, stride, size, base, rows in ((64, 8, 32, 0, 128),
                                        (32, 16, 64, 128, 32),
                                        (16, 32, 128, 160, 8),
                                        (8, 64, 256, 168, 2)):
        wh = _anchor_sizes(size)
        aw, ah = wh[a, 0], wh[a, 1]
        c1 = np.where(j == 2, aw, np.where(j == 3, ah, 0.0)).astype(np.float32)
        half = np.where(j % 2 == 0, aw, ah).astype(np.float32) * np.float32(0.5)
        sign = np.where(j < 2, -1.0, 1.0).astype(np.float32)
        c2 = (sign * half).astype(np.float32)
        levels.append((w, int(np.log2(w)), float(stride), base, rows,
                       c1[None, :], c2[None, :]))
    return q[None, :], m0[None, :], m1[None, :], mx[None, :], my[None, :], levels


_Q, _M0, _M1, _MX, _MY, _LEVELS = _build_tables()
_F32TAB = np.concatenate([_M0, _M1, _MX, _MY]
                         + [lv[5] for lv in _LEVELS]
                         + [lv[6] for lv in _LEVELS], axis=0)


def _anchor_kernel(q_ref, tab_ref, out1_ref, out2_ref):
    q = q_ref[0:1, :]
    m0, m1 = tab_ref[0:1, :], tab_ref[1:2, :]
    mx, my = tab_ref[2:3, :], tab_ref[3:4, :]
    for lv, (w, log2w, stride, base, rows, _, _) in enumerate(_LEVELS):
        g = jax.lax.broadcasted_iota(jnp.int32, (rows, _LANES), 0)
        c = g * 32 + q
        x = jnp.bitwise_and(c, w - 1)
        y = jnp.right_shift(c, log2w)
        cx = (x.astype(jnp.float32) + 0.5) * stride
        cy = (y.astype(jnp.float32) + 0.5) * stride
        c1, c2 = tab_ref[4 + lv:5 + lv, :], tab_ref[8 + lv:9 + lv, :]
        out1_ref[pl.ds(base, rows), :] = m0 * cx + m1 * cy + c1
        out2_ref[pl.ds(base, rows), :] = mx * cx + my * cy + c2


def kernel(feat0, feat1, feat2, feat3):
    del feat0, feat1, feat2, feat3
    out1, out2 = pl.pallas_call(
        _anchor_kernel,
        out_shape=(jax.ShapeDtypeStruct((170, _LANES), jnp.float32),
                   jax.ShapeDtypeStruct((170, _LANES), jnp.float32)),
    )(jnp.asarray(_Q), jnp.asarray(_F32TAB))
    return out1.reshape(48960, 4), out2.reshape(48960, 4)


# speedup vs baseline: 21.6786x; 21.6786x over previous
import numpy as np
import jax
import jax.numpy as jnp
from jax.experimental import pallas as pl

_RATIOS = np.array([0.5, 1.0, 2.0])
_SCALES = np.array([2 ** 0, 2 ** (1.0 / 3.0), 2 ** (2.0 / 3.0)])
_TOTAL = 48960


def _anchor_sizes(box_size):
    anchors = box_size * np.tile(_SCALES, (2, len(_RATIOS))).T
    areas = anchors[:, 0] * anchors[:, 1]
    anchors[:, 0] = np.sqrt(areas * np.repeat(_RATIOS, len(_SCALES)))
    anchors[:, 1] = anchors[:, 0] / np.repeat(_RATIOS, len(_SCALES))
    return anchors.astype(np.float32)


_LEVELS = []
_base = 0
for _w, _stride, _size in ((64, 8, 32), (32, 16, 64), (16, 32, 128), (8, 64, 256)):
    _n = _w * _w * 9
    _LEVELS.append((_w, int(np.log2(_w)), float(_stride), _base, _n,
                    [float(v) for v in _anchor_sizes(_size)[:, 0]]))
    _base += _n

_C9 = float(np.float32(1.0 / 9.0))
_C3 = float(np.float32(1.0 / 3.0))


def _anchor_kernel(out1_ref, out2_ref):
    f32, i32 = jnp.float32, jnp.int32
    for w, log2w, stride, base, n, wtab in _LEVELS:
        shape = (4, n)
        rl = jax.lax.broadcasted_iota(i32, shape, 1)
        rlf = rl.astype(f32)
        qf = jnp.floor((rlf + 0.5) * _C9)
        af = rlf - 9.0 * qf
        q = qf.astype(i32)
        x = jnp.bitwise_and(q, w - 1)
        y = jnp.right_shift(q, log2w)
        cx = (x.astype(f32) + 0.5) * stride
        cy = (y.astype(f32) + 0.5) * stride
        wt = jnp.full(shape, wtab[8], f32)
        for k in range(7, -1, -1):
            wt = jnp.where(af == float(k), wtab[k], wt)
        tf = jnp.floor((af + 0.5) * _C3)
        pw = jnp.where(tf == 0.0, 2.0, jnp.where(tf == 1.0, 1.0, 0.5))
        ht = wt * pw
        j = jax.lax.broadcasted_iota(i32, shape, 0)
        out1 = jnp.where(j == 0, cx, jnp.where(j == 1, cy,
                         jnp.where(j == 2, wt, ht)))
        hw = wt * 0.5
        hh = ht * 0.5
        out2 = jnp.where(j == 0, cx - hw, jnp.where(j == 1, cy - hh,
                         jnp.where(j == 2, cx + hw, cy + hh)))
        out1_ref[:, pl.ds(base, n)] = out1
        out2_ref[:, pl.ds(base, n)] = out2


def kernel(feat0, feat1, feat2, feat3):
    del feat0, feat1, feat2, feat3
    o1t, o2t = pl.pallas_call(
        _anchor_kernel,
        out_shape=(jax.ShapeDtypeStruct((4, _TOTAL), jnp.float32),
                   jax.ShapeDtypeStruct((4, _TOTAL), jnp.float32)),
    )()
    return jnp.transpose(o1t), jnp.transpose(o2t)


# speedup vs baseline: 44.7560x; 2.0645x over previous
import numpy as np
import jax
import jax.numpy as jnp
from jax.experimental import pallas as pl

_RATIOS = np.array([0.5, 1.0, 2.0])
_SCALES = np.array([2 ** 0, 2 ** (1.0 / 3.0), 2 ** (2.0 / 3.0)])
_TOTAL = 48960
_CHUNK = 1152


def _anchor_sizes(box_size):
    anchors = box_size * np.tile(_SCALES, (2, len(_RATIOS))).T
    areas = anchors[:, 0] * anchors[:, 1]
    anchors[:, 0] = np.sqrt(areas * np.repeat(_RATIOS, len(_SCALES)))
    anchors[:, 1] = anchors[:, 0] / np.repeat(_RATIOS, len(_SCALES))
    return anchors.astype(np.float32)


_LEVELS = []
_lane_base = 0
for _w, _stride, _size in ((64, 8, 32), (32, 16, 64), (16, 32, 128), (8, 64, 256)):
    _n = _w * _w * 9
    _LEVELS.append((_w, float(_stride), _lane_base, _n,
                    [float(v) for v in _anchor_sizes(_size)[:, 0]]))
    _lane_base += _n

_C9 = float(np.float32(1.0 / 9.0))
_C3 = float(np.float32(1.0 / 3.0))


def _anchor_kernel(out1_ref, out2_ref):
    f32, i32 = jnp.float32, jnp.int32
    for w, stride, base, n, wtab in _LEVELS:
        width = min(n, _CHUNK)
        shape = (4, width)
        lp = jax.lax.broadcasted_iota(i32, shape, 1).astype(f32)
        qf = jnp.floor((lp + 0.5) * _C9)
        af = lp - 9.0 * qf
        yq = jnp.floor(qf * (1.0 / w))
        xf = qf - yq * w
        cx = xf * stride + 0.5 * stride
        cy0 = yq * stride + 0.5 * stride
        wt = jnp.full(shape, wtab[8], f32)
        for k in range(7, -1, -1):
            wt = jnp.where(af == float(k), wtab[k], wt)
        tf = jnp.floor((af + 0.5) * _C3)
        pw = jnp.where(tf == 0.0, 2.0, jnp.where(tf == 1.0, 1.0, 0.5))
        ht = wt * pw
        j = jax.lax.broadcasted_iota(i32, shape, 0)
        even = (j & 1) == 0
        low = j < 2
        wh = jnp.where(even, wt, ht)
        habs = jnp.where(even, wt * 0.5, ht * 0.5)
        hsgn = jnp.where(low, -habs, habs)
        dy_step = stride * (width // 9) / w
        for i in range(n // width):
            cyv = cy0 + dy_step * i
            bc = jnp.where(even, cx, cyv)
            out1_ref[:, pl.ds(base + i * width, width)] = jnp.where(low, bc, wh)
            out2_ref[:, pl.ds(base + i * width, width)] = bc + hsgn


def kernel(feat0, feat1, feat2, feat3):
    del feat0, feat1, feat2, feat3
    o1t, o2t = pl.pallas_call(
        _anchor_kernel,
        out_shape=(jax.ShapeDtypeStruct((4, _TOTAL), jnp.float32),
                   jax.ShapeDtypeStruct((4, _TOTAL), jnp.float32)),
    )()
    return jnp.transpose(o1t), jnp.transpose(o2t)


# speedup vs baseline: 49.3153x; 1.1019x over previous
import numpy as np
import jax
import jax.numpy as jnp
from jax.experimental import pallas as pl
from jax.experimental.pallas import tpu as pltpu

_RATIOS = np.array([0.5, 1.0, 2.0])
_SCALES = np.array([2 ** 0, 2 ** (1.0 / 3.0), 2 ** (2.0 / 3.0)])
_TOTAL = 48960
_CHUNK = 1152


def _anchor_sizes(box_size):
    anchors = box_size * np.tile(_SCALES, (2, len(_RATIOS))).T
    areas = anchors[:, 0] * anchors[:, 1]
    anchors[:, 0] = np.sqrt(areas * np.repeat(_RATIOS, len(_SCALES)))
    anchors[:, 1] = anchors[:, 0] / np.repeat(_RATIOS, len(_SCALES))
    return anchors.astype(np.float32)


_LEVELS = []
_lane_base = 0
for _w, _stride, _size in ((64, 8, 32), (32, 16, 64), (16, 32, 128), (8, 64, 256)):
    _n = _w * _w * 9
    _LEVELS.append((_w, float(_stride), _lane_base, _n,
                    [float(v) for v in _anchor_sizes(_size)[:, 0]]))
    _lane_base += _n

_C9 = float(np.float32(1.0 / 9.0))
_C3 = float(np.float32(1.0 / 3.0))


_GROUPS = ((0, 18432), (18432, 18432), (36864, 9216), (46080, 2880))


def _anchor_kernel(out1_ref, out2_ref, s1, s2, sem):
    f32, i32 = jnp.float32, jnp.int32

    def _start_group(g):
        lo, width = _GROUPS[g]
        for idx, (s, o) in enumerate(((s1, out1_ref), (s2, out2_ref))):
            pltpu.make_async_copy(s.at[:, pl.ds(lo, width)],
                                  o.at[:, pl.ds(lo, width)],
                                  sem.at[2 * g + idx]).start()

    done = 0
    next_group = 0
    for w, stride, base, n, wtab in _LEVELS:
        width = min(n, _CHUNK)
        shape = (4, width)
        lp = jax.lax.broadcasted_iota(i32, shape, 1).astype(f32)
        qf = jnp.floor((lp + 0.5) * _C9)
        af = lp - 9.0 * qf
        yq = jnp.floor(qf * (1.0 / w))
        xf = qf - yq * w
        cx = xf * stride + 0.5 * stride
        cy0 = yq * stride + 0.5 * stride
        wt = jnp.full(shape, wtab[8], f32)
        for k in range(7, -1, -1):
            wt = jnp.where(af == float(k), wtab[k], wt)
        tf = jnp.floor((af + 0.5) * _C3)
        pw = jnp.where(tf == 0.0, 2.0, jnp.where(tf == 1.0, 1.0, 0.5))
        ht = wt * pw
        j = jax.lax.broadcasted_iota(i32, shape, 0)
        even = (j & 1) == 0
        low = j < 2
        wh = jnp.where(even, wt, ht)
        habs = jnp.where(even, wt * 0.5, ht * 0.5)
        hsgn = jnp.where(low, -habs, habs)
        dy_step = stride * (width // 9) / w
        for i in range(n // width):
            cyv = cy0 + dy_step * i
            bc = jnp.where(even, cx, cyv)
            s1[:, pl.ds(base + i * width, width)] = jnp.where(low, bc, wh)
            s2[:, pl.ds(base + i * width, width)] = bc + hsgn
            done += width
            while next_group < len(_GROUPS) and \
                    done >= _GROUPS[next_group][0] + _GROUPS[next_group][1]:
                _start_group(next_group)
                next_group += 1
    for g in range(len(_GROUPS)):
        lo, width = _GROUPS[g]
        for idx, (s, o) in enumerate(((s1, out1_ref), (s2, out2_ref))):
            pltpu.make_async_copy(s.at[:, pl.ds(lo, width)],
                                  o.at[:, pl.ds(lo, width)],
                                  sem.at[2 * g + idx]).wait()


def kernel(feat0, feat1, feat2, feat3):
    del feat0, feat1, feat2, feat3
    o1t, o2t = pl.pallas_call(
        _anchor_kernel,
        out_shape=(jax.ShapeDtypeStruct((4, _TOTAL), jnp.float32),
                   jax.ShapeDtypeStruct((4, _TOTAL), jnp.float32)),
        out_specs=(pl.BlockSpec(memory_space=pl.ANY),
                   pl.BlockSpec(memory_space=pl.ANY)),
        scratch_shapes=[pltpu.VMEM((4, _TOTAL), jnp.float32),
                        pltpu.VMEM((4, _TOTAL), jnp.float32),
                        pltpu.SemaphoreType.DMA((8,))],
    )()
    return jnp.transpose(o1t), jnp.transpose(o2t)


# speedup vs baseline: 49.7664x; 1.0091x over previous
import numpy as np
import jax
import jax.numpy as jnp
from jax.experimental import pallas as pl
from jax.experimental.pallas import tpu as pltpu

_RATIOS = np.array([0.5, 1.0, 2.0])
_SCALES = np.array([2 ** 0, 2 ** (1.0 / 3.0), 2 ** (2.0 / 3.0)])
_TOTAL = 48960
_CHUNK = 1152


def _anchor_sizes(box_size):
    anchors = box_size * np.tile(_SCALES, (2, len(_RATIOS))).T
    areas = anchors[:, 0] * anchors[:, 1]
    anchors[:, 0] = np.sqrt(areas * np.repeat(_RATIOS, len(_SCALES)))
    anchors[:, 1] = anchors[:, 0] / np.repeat(_RATIOS, len(_SCALES))
    return anchors.astype(np.float32)


_LEVELS = []
_lane_base = 0
for _w, _stride, _size in ((64, 8, 32), (32, 16, 64), (16, 32, 128), (8, 64, 256)):
    _n = _w * _w * 9
    _LEVELS.append((_w, float(_stride), _lane_base, _n,
                    [float(v) for v in _anchor_sizes(_size)[:, 0]]))
    _lane_base += _n

_C9 = float(np.float32(1.0 / 9.0))
_C3 = float(np.float32(1.0 / 3.0))


_GROUPS = ((0, 9216), (9216, 9216), (18432, 9216), (27648, 9216),
           (36864, 9216), (46080, 2880))


def _anchor_kernel(out1_ref, out2_ref, s1, s2, sem):
    f32, i32 = jnp.float32, jnp.int32

    def _start_group(g):
        lo, width = _GROUPS[g]
        for idx, (s, o) in enumerate(((s1, out1_ref), (s2, out2_ref))):
            pltpu.make_async_copy(s.at[:, pl.ds(lo, width)],
                                  o.at[:, pl.ds(lo, width)],
                                  sem.at[2 * g + idx]).start()

    done = 0
    next_group = 0
    for w, stride, base, n, wtab in _LEVELS:
        width = min(n, _CHUNK)
        shape = (4, width)
        lp = jax.lax.broadcasted_iota(i32, shape, 1).astype(f32)
        qf = jnp.floor((lp + 0.5) * _C9)
        af = lp - 9.0 * qf
        yq = jnp.floor(qf * (1.0 / w))
        xf = qf - yq * w
        cx = xf * stride + 0.5 * stride
        cy0 = yq * stride + 0.5 * stride
        wt = jnp.full(shape, wtab[8], f32)
        for k in range(7, -1, -1):
            wt = jnp.where(af == float(k), wtab[k], wt)
        tf = jnp.floor((af + 0.5) * _C3)
        pw = jnp.where(tf == 0.0, 2.0, jnp.where(tf == 1.0, 1.0, 0.5))
        ht = wt * pw
        j = jax.lax.broadcasted_iota(i32, shape, 0)
        even = (j & 1) == 0
        low = j < 2
        wh = jnp.where(even, wt, ht)
        habs = jnp.where(even, wt * 0.5, ht * 0.5)
        hsgn = jnp.where(low, -habs, habs)
        dy_step = stride * (width // 9) / w
        for i in range(n // width):
            cyv = cy0 + dy_step * i
            bc = jnp.where(even, cx, cyv)
            s1[:, pl.ds(base + i * width, width)] = jnp.where(low, bc, wh)
            s2[:, pl.ds(base + i * width, width)] = bc + hsgn
            done += width
            while next_group < len(_GROUPS) and \
                    done >= _GROUPS[next_group][0] + _GROUPS[next_group][1]:
                _start_group(next_group)
                next_group += 1
    for g in range(len(_GROUPS)):
        lo, width = _GROUPS[g]
        for idx, (s, o) in enumerate(((s1, out1_ref), (s2, out2_ref))):
            pltpu.make_async_copy(s.at[:, pl.ds(lo, width)],
                                  o.at[:, pl.ds(lo, width)],
                                  sem.at[2 * g + idx]).wait()


def kernel(feat0, feat1, feat2, feat3):
    del feat0, feat1, feat2, feat3
    o1t, o2t = pl.pallas_call(
        _anchor_kernel,
        out_shape=(jax.ShapeDtypeStruct((4, _TOTAL), jnp.float32),
                   jax.ShapeDtypeStruct((4, _TOTAL), jnp.float32)),
        out_specs=(pl.BlockSpec(memory_space=pl.ANY),
                   pl.BlockSpec(memory_space=pl.ANY)),
        scratch_shapes=[pltpu.VMEM((4, _TOTAL), jnp.float32),
                        pltpu.VMEM((4, _TOTAL), jnp.float32),
                        pltpu.SemaphoreType.DMA((12,))],
    )()
    return jnp.transpose(o1t), jnp.transpose(o2t)
